# Initial kernel scaffold; baseline (speedup 1.0000x reference)
import jax
import jax.numpy as jnp
from jax.experimental import pallas as pl


def kernel(X, weight):
    raise NotImplementedError("write your pallas kernel here")


# speedup vs baseline: 1.8377x; 1.8377x over previous
import functools

import jax
import jax.numpy as jnp
from jax import lax
from jax.experimental import pallas as pl
from jax.experimental.pallas import tpu as pltpu
from jax.experimental.pallas import tpu_sc as plsc

_EMBED_DIM = 64
_NUM_CORES = 2
_NUM_SUBCORES = 16
_NUM_WORKERS = _NUM_CORES * _NUM_SUBCORES
_CHUNK = 128


@functools.lru_cache(maxsize=None)
def _build_gather(n_chunk: int, n_class: int):
    mesh = plsc.VectorSubcoreMesh(core_axis_name="c", subcore_axis_name="s")

    @functools.partial(
        pl.kernel,
        out_type=jax.ShapeDtypeStruct(
            (_NUM_WORKERS, n_chunk, _CHUNK, _EMBED_DIM), jnp.float32
        ),
        mesh=mesh,
        compiler_params=pltpu.CompilerParams(use_tc_tiling_on_sc=False),
        scratch_types=[
            pltpu.VMEM((n_chunk, _CHUNK), jnp.int32),
            pltpu.VMEM((2, _CHUNK, _EMBED_DIM), jnp.float32),
            pltpu.SemaphoreType.DMA,
            pltpu.SemaphoreType.DMA,
        ],
    )
    def gather_kernel(idx_hbm, table_hbm, out_hbm, idx_v, rows_v, sem0, sem1):
        wid = lax.axis_index("s") * _NUM_CORES + lax.axis_index("c")
        pltpu.sync_copy(idx_hbm.at[wid], idx_v)
        pltpu.async_copy(table_hbm.at[idx_v.at[0]], rows_v.at[0], sem0)

        def body(g, carry):
            c0 = 2 * g
            pltpu.async_copy(table_hbm.at[idx_v.at[c0 + 1]], rows_v.at[1], sem1)
            pltpu.make_async_copy(
                table_hbm.at[idx_v.at[c0]], rows_v.at[0], sem0
            ).wait()
            pltpu.sync_copy(rows_v.at[0], out_hbm.at[wid, c0])

            @pl.when(c0 + 2 < n_chunk)
            def _():
                pltpu.async_copy(table_hbm.at[idx_v.at[c0 + 2]], rows_v.at[0], sem0)

            pltpu.make_async_copy(
                table_hbm.at[idx_v.at[c0 + 1]], rows_v.at[1], sem1
            ).wait()
            pltpu.sync_copy(rows_v.at[1], out_hbm.at[wid, c0 + 1])
            return carry

        lax.fori_loop(0, n_chunk // 2, body, 0)

    return gather_kernel


def kernel(X, weight):
    batch, hist = X.shape
    n_total = batch * hist
    idx = X.reshape(-1).astype(jnp.int32)
    block = _NUM_WORKERS * _CHUNK
    pad = (-n_total) % block
    if pad:
        idx = jnp.concatenate([idx, jnp.zeros((pad,), jnp.int32)])
    n_chunk = (n_total + pad) // block
    idx3 = idx.reshape(_NUM_WORKERS, n_chunk, _CHUNK)
    out = _build_gather(n_chunk, weight.shape[0])(idx3, weight)
    out = out.reshape(-1, _EMBED_DIM)
    if pad:
        out = out[:n_total]
    return out.reshape(batch, hist, _EMBED_DIM)


# speedup vs baseline: 1.8715x; 1.0184x over previous
import functools

import jax
import jax.numpy as jnp
from jax import lax
from jax.experimental import pallas as pl
from jax.experimental.pallas import tpu as pltpu
from jax.experimental.pallas import tpu_sc as plsc

_EMBED_DIM = 64
_NUM_CORES = 2
_NUM_SUBCORES = 16
_NUM_WORKERS = _NUM_CORES * _NUM_SUBCORES
_CHUNK = 128
_NBUF = 8


@functools.lru_cache(maxsize=None)
def _build_gather(n_chunk: int, n_class: int):
    mesh = plsc.VectorSubcoreMesh(core_axis_name="c", subcore_axis_name="s")
    assert n_chunk % _NBUF == 0

    @functools.partial(
        pl.kernel,
        out_type=jax.ShapeDtypeStruct(
            (_NUM_WORKERS, n_chunk, _CHUNK, _EMBED_DIM), jnp.float32
        ),
        mesh=mesh,
        compiler_params=pltpu.CompilerParams(use_tc_tiling_on_sc=False),
        scratch_types=[
            pltpu.VMEM((n_chunk, _CHUNK), jnp.int32),
            pltpu.VMEM((_NBUF, _CHUNK, _EMBED_DIM), jnp.float32),
            pltpu.SemaphoreType.DMA((_NBUF,)),
            pltpu.SemaphoreType.DMA((_NBUF,)),
        ],
    )
    def gather_kernel(idx_hbm, table_hbm, out_hbm, idx_v, rows_v, sem_in, sem_out):
        wid = lax.axis_index("s") * _NUM_CORES + lax.axis_index("c")
        pltpu.sync_copy(idx_hbm.at[wid], idx_v)
        for b in range(_NBUF):
            pltpu.async_copy(table_hbm.at[idx_v.at[b]], rows_v.at[b], sem_in.at[b])

        def body(g, carry):
            c_base = g * _NBUF
            for b in range(_NBUF):
                c = c_base + b
                pltpu.make_async_copy(
                    table_hbm.at[idx_v.at[c]], rows_v.at[b], sem_in.at[b]
                ).wait()
                pltpu.async_copy(rows_v.at[b], out_hbm.at[wid, c], sem_out.at[b])
            for b in range(_NBUF):
                c_next = c_base + _NBUF + b

                @pl.when(c_next < n_chunk)
                def _():
                    pltpu.make_async_copy(
                        rows_v.at[b], out_hbm.at[wid, c_base + b], sem_out.at[b]
                    ).wait()
                    pltpu.async_copy(
                        table_hbm.at[idx_v.at[c_next]], rows_v.at[b], sem_in.at[b]
                    )

            return carry

        lax.fori_loop(0, n_chunk // _NBUF, body, 0)
        for b in range(_NBUF):
            pltpu.make_async_copy(
                rows_v.at[b], out_hbm.at[wid, n_chunk - _NBUF + b], sem_out.at[b]
            ).wait()

    return gather_kernel


def kernel(X, weight):
    batch, hist = X.shape
    n_total = batch * hist
    idx = X.reshape(-1).astype(jnp.int32)
    block = _NUM_WORKERS * _CHUNK
    pad = (-n_total) % block
    if pad:
        idx = jnp.concatenate([idx, jnp.zeros((pad,), jnp.int32)])
    n_chunk = (n_total + pad) // block
    idx3 = idx.reshape(_NUM_WORKERS, n_chunk, _CHUNK)
    out = _build_gather(n_chunk, weight.shape[0])(idx3, weight)
    out = out.reshape(-1, _EMBED_DIM)
    if pad:
        out = out[:n_total]
    return out.reshape(batch, hist, _EMBED_DIM)
